# baseline (speedup 1.0000x reference)
import functools

import jax
import jax.numpy as jnp
from jax import lax
from jax.experimental import pallas as pl
from jax.experimental.pallas import tpu as pltpu
from jax.experimental.pallas import tpu_sc as plsc

_S = 196
_B = 64
_C = 2048
_CTC = 1536
_CSC = 512
_NS = 16
_SCHUNK = 14
_WBLK = 128
_SC_NRG = 7
_SC_NCC = 4
_SC_SPG = _S // _SC_NRG
_SC_PASSES = 4
_SC_SPP = _SC_SPG // _SC_PASSES
_SC_PROWS = _SC_SPP * _B


def _sc_maxpool_body(feat_ref, out_ref, buf0, buf1, acc_ref, sem0, sem1):
    wid = lax.axis_index("s") * 2 + lax.axis_index("c")
    rg = wid // _SC_NCC
    cc = lax.rem(wid, _SC_NCC)
    c0 = _CTC + cc * 128
    r0 = rg * (_SC_SPG * _B)

    @pl.when(wid < _SC_NRG * _SC_NCC)
    def _active():
        def slab(g):
            return feat_ref.at[pl.ds(r0 + g * _SC_PROWS, _SC_PROWS),
                               pl.ds(c0, 128)]

        bufs = [buf0, buf1]
        sems = [sem0, sem1]
        copies = [pltpu.async_copy(slab(0), buf0, sem0),
                  pltpu.async_copy(slab(1), buf1, sem1)]

        def fold(buf, first):
            def body(b, _):
                for v in range(8):
                    cs = pl.ds(v * 16, 16)
                    if first:
                        x = buf[b, cs]
                        lo = 1
                    else:
                        x = acc_ref[b, cs]
                        lo = 0
                    for s_ in range(lo, _SC_SPP):
                        x = jnp.maximum(x, buf[s_ * _B + b, cs])
                    acc_ref[b, cs] = x
                return 0

            lax.fori_loop(0, _B, body, 0)

        for g in range(_SC_PASSES):
            copies[g % 2].wait()
            fold(bufs[g % 2], first=(g == 0))
            if g + 2 < _SC_PASSES:
                copies[g % 2] = pltpu.async_copy(slab(g + 2), bufs[g % 2],
                                                 sems[g % 2])

        pltpu.sync_copy(acc_ref, out_ref.at[rg, :, pl.ds(cc * 128, 128)])


_sc_maxpool = functools.partial(
    pl.kernel,
    out_type=jax.ShapeDtypeStruct((_SC_NRG, _B, _CSC), jnp.float32),
    mesh=plsc.VectorSubcoreMesh(core_axis_name="c", subcore_axis_name="s"),
    scratch_types=[
        pltpu.VMEM((_SC_PROWS, 128), jnp.float32),
        pltpu.VMEM((_SC_PROWS, 128), jnp.float32),
        pltpu.VMEM((_B, 128), jnp.float32),
        pltpu.SemaphoreType.DMA,
        pltpu.SemaphoreType.DMA,
    ],
)(_sc_maxpool_body)


def _fused(feat_ref, x0_ref, A_ref, Wl1_ref, bl1_ref, Wr1_ref,
           Wl2_ref, bl2_ref, Wr2_ref, out_ref, x2sc_ref, x1_ref, agg1_ref,
           facc_ref, x2_ref):
    i = pl.program_id(0)

    @pl.when(i == 0)
    def _prologue():
        A = A_ref[...]
        rs_col = jnp.sum(A, axis=1, keepdims=True)
        rs_row = jnp.sum(A.T, axis=0, keepdims=True)
        adj = lax.rsqrt(rs_col) * A.T * lax.rsqrt(rs_row)
        maskf = (adj.astype(jnp.int32) != 0).astype(jnp.float32)
        cnt = jnp.maximum(jnp.sum(maskf, axis=0, keepdims=True), 1.0)
        maskn = maskf / cnt
        x0 = x0_ref[...]
        agg0 = lax.dot_general(maskn, x0, (((0,), (0,)), ((), ())),
                               preferred_element_type=jnp.float32)
        x1 = (jnp.dot(agg0, Wl1_ref[...], preferred_element_type=jnp.float32)
              + bl1_ref[...]
              + jnp.dot(x0, Wr1_ref[...], preferred_element_type=jnp.float32))
        x1 = jnp.where(x1 > 0, x1, 0.2 * x1)
        x1_ref[...] = x1
        agg1_ref[...] = lax.dot_general(maskn, x1, (((0,), (0,)), ((), ())),
                                        preferred_element_type=jnp.float32)

    slab_max = jnp.max(feat_ref[...], axis=0)

    @pl.when(i == 0)
    def _init_max():
        facc_ref[...] = slab_max

    @pl.when(i > 0)
    def _fold_max():
        facc_ref[...] = jnp.maximum(facc_ref[...], slab_max)

    x2_ref[:, pl.ds(i * _WBLK, _WBLK)] = (
        jnp.dot(agg1_ref[...], Wl2_ref[...], preferred_element_type=jnp.float32)
        + bl2_ref[...]
        + jnp.dot(x1_ref[...], Wr2_ref[...], preferred_element_type=jnp.float32))

    @pl.when(i == _NS - 1)
    def _classify():
        out_ref[...] = lax.dot_general(
            facc_ref[...], x2_ref[:, :_CTC], (((1,), (1,)), ((), ())),
            preferred_element_type=jnp.float32)
        x2sc_ref[...] = x2_ref[:, _CTC:]


def _combine(out_tc_ref, fsc_ref, x2sc_ref, out_ref):
    fsc = jnp.max(fsc_ref[...], axis=0)
    out_ref[...] = out_tc_ref[...] + lax.dot_general(
        fsc, x2sc_ref[...], (((1,), (1,)), ((), ())),
        preferred_element_type=jnp.float32)


def kernel(feature, inp, A, Wl1, bl1, Wr1, Wl2, bl2, Wr2):
    B, C = feature.shape[0], feature.shape[1]
    N = A.shape[0]
    H1 = Wl1.shape[1]
    featT = jnp.transpose(feature, (2, 3, 0, 1)).reshape(_S, B, C)
    feat2 = featT.reshape(_S * B, C)
    x0 = inp[0]
    nslab = _S // _SCHUNK

    fsc = _sc_maxpool(feat2)

    out_tc, x2sc = pl.pallas_call(
        _fused,
        grid=(_NS,),
        in_specs=[
            pl.BlockSpec((_SCHUNK, B, _CTC), lambda i: (jnp.minimum(i, nslab - 1), 0, 0)),
            pl.BlockSpec(x0.shape, lambda i: (0, 0)),
            pl.BlockSpec(A.shape, lambda i: (0, 0)),
            pl.BlockSpec(Wl1.shape, lambda i: (0, 0)),
            pl.BlockSpec((1, H1), lambda i: (0, 0)),
            pl.BlockSpec(Wr1.shape, lambda i: (0, 0)),
            pl.BlockSpec((Wl2.shape[0], _WBLK), lambda i: (0, i)),
            pl.BlockSpec((1, _WBLK), lambda i: (0, i)),
            pl.BlockSpec((Wl2.shape[0], _WBLK), lambda i: (0, i)),
        ],
        out_specs=[
            pl.BlockSpec((B, N), lambda i: (0, 0)),
            pl.BlockSpec((N, _CSC), lambda i: (0, 0)),
        ],
        out_shape=[
            jax.ShapeDtypeStruct((B, N), jnp.float32),
            jax.ShapeDtypeStruct((N, _CSC), jnp.float32),
        ],
        scratch_shapes=[
            pltpu.VMEM((N, H1), jnp.float32),
            pltpu.VMEM((N, H1), jnp.float32),
            pltpu.VMEM((B, _CTC), jnp.float32),
            pltpu.VMEM((N, _C), jnp.float32),
        ],
        compiler_params=pltpu.CompilerParams(
            dimension_semantics=("arbitrary",),
        ),
    )(featT, x0, A, Wl1, bl1.reshape(1, -1), Wr1,
      Wl2, bl2.reshape(1, -1), Wr2)

    return pl.pallas_call(
        _combine,
        out_shape=jax.ShapeDtypeStruct((B, N), jnp.float32),
    )(out_tc, fsc, x2sc)


# speedup vs baseline: 1.4650x; 1.4650x over previous
import jax
import jax.numpy as jnp
from jax import lax
from jax.experimental import pallas as pl
from jax.experimental.pallas import tpu as pltpu

_S = 196
_NS = 16
_SCHUNK = 14
_WBLK = 128


def _fused(feat_ref, featb_ref, x0_ref, A_ref, Wl1_ref, bl1_ref, Wr1_ref,
           Wl2_ref, bl2_ref, Wr2_ref, out_ref, x1_ref, agg1_ref,
           facc_ref, x2_ref):
    i = pl.program_id(0)

    @pl.when(i == 0)
    def _prologue():
        A = A_ref[...]
        rs_col = jnp.sum(A, axis=1, keepdims=True)
        rs_row = jnp.sum(A.T, axis=0, keepdims=True)
        adj = lax.rsqrt(rs_col) * A.T * lax.rsqrt(rs_row)
        maskf = (adj.astype(jnp.int32) != 0).astype(jnp.float32)
        cnt = jnp.maximum(jnp.sum(maskf, axis=0, keepdims=True), 1.0)
        maskn = maskf / cnt
        x0 = x0_ref[...]
        agg0 = lax.dot_general(maskn, x0, (((0,), (0,)), ((), ())),
                               preferred_element_type=jnp.float32)
        x1 = (jnp.dot(agg0, Wl1_ref[...], preferred_element_type=jnp.float32)
              + bl1_ref[...]
              + jnp.dot(x0, Wr1_ref[...], preferred_element_type=jnp.float32))
        x1 = jnp.where(x1 > 0, x1, 0.2 * x1)
        x1_ref[...] = x1
        agg1_ref[...] = lax.dot_general(maskn, x1, (((0,), (0,)), ((), ())),
                                        preferred_element_type=jnp.float32)

    half = facc_ref.shape[1] // 2
    slab_max_a = jnp.max(feat_ref[...], axis=0)
    slab_max_b = jnp.max(featb_ref[...], axis=0)

    @pl.when(i == 0)
    def _init_max():
        facc_ref[:, :half] = slab_max_a
        facc_ref[:, half:] = slab_max_b

    @pl.when(i > 0)
    def _fold_max():
        facc_ref[:, :half] = jnp.maximum(facc_ref[:, :half], slab_max_a)
        facc_ref[:, half:] = jnp.maximum(facc_ref[:, half:], slab_max_b)

    x2_ref[:, pl.ds(i * _WBLK, _WBLK)] = (
        jnp.dot(agg1_ref[...], Wl2_ref[...], preferred_element_type=jnp.float32)
        + bl2_ref[...]
        + jnp.dot(x1_ref[...], Wr2_ref[...], preferred_element_type=jnp.float32))

    @pl.when(i == _NS - 1)
    def _classify():
        out_ref[...] = lax.dot_general(
            facc_ref[...], x2_ref[...], (((1,), (1,)), ((), ())),
            preferred_element_type=jnp.float32)


def kernel(feature, inp, A, Wl1, bl1, Wr1, Wl2, bl2, Wr2):
    B, C = feature.shape[0], feature.shape[1]
    N = A.shape[0]
    H1 = Wl1.shape[1]
    featT = jnp.transpose(feature, (2, 3, 0, 1)).reshape(_S, B, C)
    x0 = inp[0]
    nslab = _S // _SCHUNK
    return pl.pallas_call(
        _fused,
        grid=(_NS,),
        in_specs=[
            pl.BlockSpec((_SCHUNK, B, C // 2), lambda i: (jnp.minimum(i, nslab - 1), 0, 0)),
            pl.BlockSpec((_SCHUNK, B, C // 2), lambda i: (jnp.minimum(i, nslab - 1), 0, 1)),
            pl.BlockSpec(x0.shape, lambda i: (0, 0)),
            pl.BlockSpec(A.shape, lambda i: (0, 0)),
            pl.BlockSpec(Wl1.shape, lambda i: (0, 0)),
            pl.BlockSpec((1, H1), lambda i: (0, 0)),
            pl.BlockSpec(Wr1.shape, lambda i: (0, 0)),
            pl.BlockSpec((Wl2.shape[0], _WBLK), lambda i: (0, i)),
            pl.BlockSpec((1, _WBLK), lambda i: (0, i)),
            pl.BlockSpec((Wl2.shape[0], _WBLK), lambda i: (0, i)),
        ],
        out_specs=pl.BlockSpec((B, N), lambda i: (0, 0)),
        out_shape=jax.ShapeDtypeStruct((B, N), jnp.float32),
        scratch_shapes=[
            pltpu.VMEM((N, H1), jnp.float32),
            pltpu.VMEM((N, H1), jnp.float32),
            pltpu.VMEM((B, C), jnp.float32),
            pltpu.VMEM((N, C), jnp.float32),
        ],
        compiler_params=pltpu.CompilerParams(
            dimension_semantics=("arbitrary",),
        ),
    )(featT, featT, x0, A, Wl1, bl1.reshape(1, -1), Wr1,
      Wl2, bl2.reshape(1, -1), Wr2)
